# baseline (speedup 1.0000x reference)
import functools

import jax
import jax.numpy as jnp
from jax import lax
from jax.experimental import pallas as pl
from jax.experimental.pallas import tpu as pltpu
from jax.experimental.pallas import tpu_sc as plsc

_B, _C, _H, _W = 8, 24, 256, 256
_NCLS = 20
_NBOX = 4
_RPT = 8
_TASKS_PER_IMG = _H // _RPT
_LANES = 4
_T = _TASKS_PER_IMG // _LANES
_PW = _W + 32
_CH_STRIDE = 10 * _W
_IMG = _C * _H * _W


def _sc_nms(x_hbm, o_hbm, cls_v, box_v, probs_v, mask_v, sem_in, sem_out):
    cid = lax.axis_index("c")
    sid = lax.axis_index("s")
    wid = sid * 2 + cid
    img = wid // _LANES
    lane = wid % _LANES

    zero16 = jnp.zeros((16,), jnp.float32)

    for r in range(10):
        probs_v[pl.ds(r * _PW, 16)] = zero16
        probs_v[pl.ds(r * _PW + _W + 16, 16)] = zero16

    img_base = img * _IMG
    last = _TASKS_PER_IMG - 1

    def task(t, _carry):
        chunk = lane + _LANES * t
        r0 = chunk * _RPT

        def fire_cls(src_row, dst_slot, nrows):
            copies = []
            for ch in range(_NCLS):
                src = img_base + ch * _H * _W + src_row * _W
                src = pl.multiple_of(src, 256)
                copies.append(pltpu.make_async_copy(
                    x_hbm.at[pl.ds(src, nrows * _W)],
                    cls_v.at[pl.ds(ch * _CH_STRIDE + dst_slot * _W, nrows * _W)],
                    sem_in))
            for cp in copies:
                cp.start()
            for cp in copies:
                cp.wait()

        @pl.when(chunk == 0)
        def _():
            fire_cls(r0, 1, 9)
        @pl.when(chunk == last)
        def _():
            fire_cls(r0 - 1, 0, 9)
        @pl.when((chunk != 0) & (chunk != last))
        def _():
            fire_cls(r0 - 1, 0, 10)

        box_copies = []
        for bc in range(_NBOX):
            src = img_base + (_NCLS + bc) * _H * _W + r0 * _W
            src = pl.multiple_of(src, 256)
            box_copies.append(pltpu.make_async_copy(
                x_hbm.at[pl.ds(src, _RPT * _W)],
                box_v.at[pl.ds(bc * _RPT * _W, _RPT * _W)],
                sem_in))
        for cp in box_copies:
            cp.start()
        for cp in box_copies:
            cp.wait()

        def probs_row(r, _):
            for j in range(16):
                off = r * _W + j * 16
                acc = cls_v[pl.ds(off, 16)]
                for ch in range(1, _NCLS):
                    acc = jnp.maximum(acc, cls_v[pl.ds(ch * _CH_STRIDE + off, 16)])
                probs_v[pl.ds(r * _PW + 16 + j * 16, 16)] = acc
            return 0
        lax.fori_loop(0, 10, probs_row, 0)

        def zero_probs_row(r):
            for j in range(16):
                probs_v[pl.ds(r * _PW + 16 + j * 16, 16)] = zero16
        @pl.when(chunk == 0)
        def _():
            zero_probs_row(0)
        @pl.when(chunk == last)
        def _():
            zero_probs_row(9)

        def mask_row(i, _):
            ra = i * _PW
            rb = (i + 1) * _PW
            rc = (i + 2) * _PW
            for j in range(16):
                cb = 16 + j * 16
                n00 = probs_v[pl.ds(ra + cb - 1, 16)]
                n01 = probs_v[pl.ds(ra + cb, 16)]
                n02 = probs_v[pl.ds(ra + cb + 1, 16)]
                n10 = probs_v[pl.ds(rb + cb - 1, 16)]
                c0 = probs_v[pl.ds(rb + cb, 16)]
                n12 = probs_v[pl.ds(rb + cb + 1, 16)]
                n20 = probs_v[pl.ds(rc + cb - 1, 16)]
                n21 = probs_v[pl.ds(rc + cb, 16)]
                n22 = probs_v[pl.ds(rc + cb + 1, 16)]
                strict = jnp.maximum(jnp.maximum(n00, n01),
                                     jnp.maximum(n02, n10))
                nonstrict = jnp.maximum(jnp.maximum(n12, n20),
                                        jnp.maximum(n21, n22))
                m = (c0 > strict) & (c0 >= nonstrict)
                mask_v[pl.ds(i * _W + j * 16, 16)] = jnp.where(m, 1.0, 0.0)
            return 0
        lax.fori_loop(0, _RPT, mask_row, 0)

        def mul_cls(ci, _):
            ch = ci // _RPT
            i = ci % _RPT
            for j in range(16):
                moff = i * _W + j * 16
                off = ch * _CH_STRIDE + (i + 1) * _W + j * 16
                cls_v[pl.ds(off, 16)] = (cls_v[pl.ds(off, 16)]
                                         * mask_v[pl.ds(moff, 16)])
            return 0
        lax.fori_loop(0, _NCLS * _RPT, mul_cls, 0)

        def mul_box(ci, _):
            for j in range(16):
                off = ci * _W + j * 16
                box_v[pl.ds(off, 16)] = (box_v[pl.ds(off, 16)]
                                         * mask_v[pl.ds(off % (_RPT * _W), 16)])
            return 0
        lax.fori_loop(0, _NBOX * _RPT, mul_box, 0)

        out_copies = []
        for ch in range(_NCLS):
            dst = img_base + ch * _H * _W + r0 * _W
            dst = pl.multiple_of(dst, 256)
            out_copies.append(pltpu.make_async_copy(
                cls_v.at[pl.ds(ch * _CH_STRIDE + _W, _RPT * _W)],
                o_hbm.at[pl.ds(dst, _RPT * _W)],
                sem_out))
        for bc in range(_NBOX):
            dst = img_base + (_NCLS + bc) * _H * _W + r0 * _W
            dst = pl.multiple_of(dst, 256)
            out_copies.append(pltpu.make_async_copy(
                box_v.at[pl.ds(bc * _RPT * _W, _RPT * _W)],
                o_hbm.at[pl.ds(dst, _RPT * _W)],
                sem_out))
        for cp in out_copies:
            cp.start()
        for cp in out_copies:
            cp.wait()
        return 0

    lax.fori_loop(0, _T, task, 0)


def kernel(points):
    flat = points.reshape(-1)
    mesh = plsc.VectorSubcoreMesh(core_axis_name="c", subcore_axis_name="s")
    k = functools.partial(
        pl.kernel,
        mesh=mesh,
        out_type=jax.ShapeDtypeStruct((_B * _IMG,), jnp.float32),
        scratch_types=[
            pltpu.VMEM((_NCLS * _CH_STRIDE,), jnp.float32),
            pltpu.VMEM((_NBOX * _RPT * _W,), jnp.float32),
            pltpu.VMEM((10 * _PW,), jnp.float32),
            pltpu.VMEM((_RPT * _W,), jnp.float32),
            pltpu.SemaphoreType.DMA,
            pltpu.SemaphoreType.DMA,
        ],
    )(_sc_nms)
    return k(flat).reshape(points.shape)


# speedup vs baseline: 9.5860x; 9.5860x over previous
import jax
import jax.numpy as jnp
from jax.experimental import pallas as pl

_NUM_CLASS_CH = 20


def _nms_block(x_ref, o_ref):
    x = x_ref[...]
    probs = jnp.max(x[:, :_NUM_CLASS_CH], axis=1)
    Bb, H, W = probs.shape
    zrow = jnp.zeros((Bb, 1, W), probs.dtype)
    zcol = jnp.zeros((Bb, H, 1), probs.dtype)

    def shl(a):
        return jnp.concatenate([zcol, a[:, :, :-1]], axis=2)

    def shr(a):
        return jnp.concatenate([a[:, :, 1:], zcol], axis=2)

    up = jnp.concatenate([zrow, probs[:, :-1]], axis=1)
    down = jnp.concatenate([probs[:, 1:], zrow], axis=1)

    strict = jnp.maximum(jnp.maximum(shl(up), up),
                         jnp.maximum(shr(up), shl(probs)))
    nonstrict = jnp.maximum(jnp.maximum(shr(probs), shl(down)),
                            jnp.maximum(down, shr(down)))
    mask = ((probs > strict) & (probs >= nonstrict)).astype(x.dtype)
    o_ref[...] = x * mask[:, None, :, :]


def kernel(points):
    B, C, H, W = points.shape
    return pl.pallas_call(
        _nms_block,
        grid=(B // 2,),
        in_specs=[pl.BlockSpec((2, C, H, W), lambda b: (b, 0, 0, 0))],
        out_specs=pl.BlockSpec((2, C, H, W), lambda b: (b, 0, 0, 0)),
        out_shape=jax.ShapeDtypeStruct(points.shape, points.dtype),
    )(points)
